# baseline (speedup 1.0000x reference)
import functools

import jax
import jax.numpy as jnp
from jax import lax
from jax.experimental import pallas as pl
from jax.experimental.pallas import tpu as pltpu
from jax.experimental.pallas import tpu_sc as plsc

_NUM_CORES = 2
_NUM_SUBCORES = 16
_NUM_WORKERS = _NUM_CORES * _NUM_SUBCORES
_LANES = 16
_CHUNK = 16
_SC_SHARE = 7680
_TC_BLK = 512


def _build_sc_kernel(num_loc, K, D, sc_loc):
    per_w = sc_loc // _NUM_WORKERS
    n_chunks = per_w // _CHUNK
    out_row = D + 1

    mesh = plsc.VectorSubcoreMesh(core_axis_name="c", subcore_axis_name="s")

    @functools.partial(
        pl.kernel,
        mesh=mesh,
        compiler_params=pltpu.CompilerParams(needs_layout_passes=False),
        out_type=jax.ShapeDtypeStruct((out_row * sc_loc,), jnp.float32),
        scratch_types=[
            pltpu.VMEM((per_w, K), jnp.float32),
            pltpu.VMEM((per_w,), jnp.int32),
            pltpu.VMEM((_CHUNK, D, K), jnp.float32),
            pltpu.VMEM((_CHUNK, D, K), jnp.float32),
            pltpu.VMEM((out_row * per_w,), jnp.float32),
            pltpu.SemaphoreType.DMA,
            pltpu.SemaphoreType.DMA,
        ],
    )
    def body(dist_hbm, sims_hbm, out_hbm, sims_v, kbuf_v, db0, db1, outbuf_v,
             sem0, sem1):
        wid = lax.axis_index("s") * _NUM_CORES + lax.axis_index("c")
        base_loc = wid * per_w
        iota = lax.iota(jnp.int32, _LANES)
        dbufs = (db0, db1)
        sems = (sem0, sem1)

        def start(c):
            return pltpu.async_copy(
                dist_hbm.at[pl.ds(base_loc + c * _CHUNK, _CHUNK)],
                dbufs[c % 2],
                sems[c % 2],
            )

        pending = start(0)

        pltpu.sync_copy(sims_hbm.at[pl.ds(base_loc, per_w)], sims_v)

        def group_body(g, carry):
            l0 = g * _LANES + iota
            best_val = plsc.load_gather(
                sims_v, [l0, jnp.zeros((_LANES,), jnp.int32)]
            )
            best_k = jnp.zeros((_LANES,), jnp.int32)
            for k in range(1, K):
                v = plsc.load_gather(
                    sims_v, [l0, jnp.full((_LANES,), k, jnp.int32)]
                )
                take = v > best_val
                best_val = jnp.where(take, v, best_val)
                best_k = jnp.where(take, k, best_k)
            plsc.store_scatter(kbuf_v, [l0], best_k)
            outbuf_v[pl.ds(D * per_w + g * _LANES, _LANES)] = (
                best_k.astype(jnp.float32)
            )
            return carry

        lax.fori_loop(0, per_w // _LANES, group_body, 0)

        for c in range(n_chunks):
            nxt = start(c + 1) if c + 1 < n_chunks else None
            pending.wait()
            dbuf = dbufs[c % 2]
            for g in range(_CHUNK // _LANES):
                lb = c * _CHUNK + g * _LANES
                ks = kbuf_v[pl.ds(lb, _LANES)]
                jvec = g * _LANES + iota
                for dd in range(D):
                    val = plsc.load_gather(
                        dbuf, [jvec, jnp.full((_LANES,), dd, jnp.int32), ks]
                    )
                    outbuf_v[pl.ds(dd * per_w + lb, _LANES)] = val
            pending = nxt

        for comp in range(out_row):
            pltpu.sync_copy(
                outbuf_v.at[pl.ds(comp * per_w, per_w)],
                out_hbm.at[pl.ds(comp * sc_loc + base_loc, per_w)],
            )

    return body


def _tc_body(K, D, d_ref, s_ref, o_ref):
    s = s_ref[...]
    ik = lax.broadcasted_iota(jnp.int32, s.shape, 1)
    m = jnp.max(s, axis=-1, keepdims=True)
    am = jnp.min(jnp.where(s == m, ik, K), axis=-1)
    onehot = (ik == am[:, None]).astype(jnp.float32)
    d = d_ref[...]
    resid = jnp.sum(d * onehot[:, None, :], axis=-1)
    o_ref[...] = jnp.concatenate(
        [resid.T, am[None, :].astype(jnp.float32)], axis=0
    )


def _tc_kernel(dist_t, sims2d, start_loc):
    num_loc, D, K = dist_t.shape
    n = num_loc - start_loc
    off = start_loc // _TC_BLK
    return pl.pallas_call(
        functools.partial(_tc_body, K, D),
        grid=(n // _TC_BLK,),
        in_specs=[
            pl.BlockSpec((_TC_BLK, D, K), lambda g: (g + off, 0, 0)),
            pl.BlockSpec((_TC_BLK, K), lambda g: (g + off, 0)),
        ],
        out_specs=pl.BlockSpec((D + 1, _TC_BLK), lambda g: (0, g)),
        out_shape=jax.ShapeDtypeStruct((D + 1, n), jnp.float32),
    )(dist_t, sims2d)


def kernel(distance, similarities):
    B, i, j, K, D = distance.shape
    num_loc = B * i * j
    dist_t = jnp.transpose(distance, (0, 1, 2, 4, 3)).reshape(num_loc, D, K)
    sims2d = similarities.reshape(num_loc, K)
    sc_out = _build_sc_kernel(num_loc, K, D, _SC_SHARE)(dist_t, sims2d)
    tc_out = _tc_kernel(dist_t, sims2d, _SC_SHARE)
    out_t = jnp.concatenate(
        [sc_out.reshape(D + 1, _SC_SHARE), tc_out], axis=1
    )
    return out_t.reshape(D + 1, B, i * j).transpose(1, 2, 0)


# speedup vs baseline: 1.0089x; 1.0089x over previous
import functools

import jax
import jax.numpy as jnp
from jax import lax
from jax.experimental import pallas as pl
from jax.experimental.pallas import tpu as pltpu
from jax.experimental.pallas import tpu_sc as plsc

_NUM_CORES = 2
_NUM_SUBCORES = 16
_NUM_WORKERS = _NUM_CORES * _NUM_SUBCORES
_LANES = 16
_CHUNK = 16
_SC_SHARE = 7168
_TC_BLK = 1024


def _build_sc_kernel(num_loc, K, D, sc_loc):
    per_w = sc_loc // _NUM_WORKERS
    n_chunks = per_w // _CHUNK
    out_row = D + 1

    mesh = plsc.VectorSubcoreMesh(core_axis_name="c", subcore_axis_name="s")

    @functools.partial(
        pl.kernel,
        mesh=mesh,
        compiler_params=pltpu.CompilerParams(needs_layout_passes=False),
        out_type=jax.ShapeDtypeStruct((out_row * sc_loc,), jnp.float32),
        scratch_types=[
            pltpu.VMEM((per_w, K), jnp.float32),
            pltpu.VMEM((per_w,), jnp.int32),
            pltpu.VMEM((_CHUNK, D, K), jnp.float32),
            pltpu.VMEM((_CHUNK, D, K), jnp.float32),
            pltpu.VMEM((out_row * per_w,), jnp.float32),
            pltpu.SemaphoreType.DMA,
            pltpu.SemaphoreType.DMA,
        ],
    )
    def body(dist_hbm, sims_hbm, out_hbm, sims_v, kbuf_v, db0, db1, outbuf_v,
             sem0, sem1):
        wid = lax.axis_index("s") * _NUM_CORES + lax.axis_index("c")
        base_loc = wid * per_w
        iota = lax.iota(jnp.int32, _LANES)
        dbufs = (db0, db1)
        sems = (sem0, sem1)

        def start(c):
            return pltpu.async_copy(
                dist_hbm.at[pl.ds(base_loc + c * _CHUNK, _CHUNK)],
                dbufs[c % 2],
                sems[c % 2],
            )

        pending = start(0)

        pltpu.sync_copy(sims_hbm.at[pl.ds(base_loc, per_w)], sims_v)

        def group_body(g, carry):
            l0 = g * _LANES + iota
            best_val = plsc.load_gather(
                sims_v, [l0, jnp.zeros((_LANES,), jnp.int32)]
            )
            best_k = jnp.zeros((_LANES,), jnp.int32)
            for k in range(1, K):
                v = plsc.load_gather(
                    sims_v, [l0, jnp.full((_LANES,), k, jnp.int32)]
                )
                take = v > best_val
                best_val = jnp.where(take, v, best_val)
                best_k = jnp.where(take, k, best_k)
            plsc.store_scatter(kbuf_v, [l0], best_k)
            outbuf_v[pl.ds(D * per_w + g * _LANES, _LANES)] = (
                best_k.astype(jnp.float32)
            )
            return carry

        lax.fori_loop(0, per_w // _LANES, group_body, 0)

        for c in range(n_chunks):
            nxt = start(c + 1) if c + 1 < n_chunks else None
            pending.wait()
            dbuf = dbufs[c % 2]
            for g in range(_CHUNK // _LANES):
                lb = c * _CHUNK + g * _LANES
                ks = kbuf_v[pl.ds(lb, _LANES)]
                jvec = g * _LANES + iota
                for dd in range(D):
                    val = plsc.load_gather(
                        dbuf, [jvec, jnp.full((_LANES,), dd, jnp.int32), ks]
                    )
                    outbuf_v[pl.ds(dd * per_w + lb, _LANES)] = val
            pending = nxt

        for comp in range(out_row):
            pltpu.sync_copy(
                outbuf_v.at[pl.ds(comp * per_w, per_w)],
                out_hbm.at[pl.ds(comp * sc_loc + base_loc, per_w)],
            )

    return body


def _tc_body(K, D, d_ref, s_ref, o_ref):
    s = s_ref[...]
    ik = lax.broadcasted_iota(jnp.int32, s.shape, 1)
    m = jnp.max(s, axis=-1, keepdims=True)
    am = jnp.min(jnp.where(s == m, ik, K), axis=-1)
    onehot = (ik == am[:, None]).astype(jnp.float32)
    d = d_ref[...]
    resid = jnp.sum(d * onehot[:, None, :], axis=-1)
    o_ref[...] = jnp.concatenate(
        [resid.T, am[None, :].astype(jnp.float32)], axis=0
    )


def _tc_kernel(dist_t, sims2d, start_loc):
    num_loc, D, K = dist_t.shape
    n = num_loc - start_loc
    off = start_loc // _TC_BLK
    return pl.pallas_call(
        functools.partial(_tc_body, K, D),
        grid=(n // _TC_BLK,),
        in_specs=[
            pl.BlockSpec((_TC_BLK, D, K), lambda g: (g + off, 0, 0)),
            pl.BlockSpec((_TC_BLK, K), lambda g: (g + off, 0)),
        ],
        out_specs=pl.BlockSpec((D + 1, _TC_BLK), lambda g: (0, g)),
        out_shape=jax.ShapeDtypeStruct((D + 1, n), jnp.float32),
    )(dist_t, sims2d)


def kernel(distance, similarities):
    B, i, j, K, D = distance.shape
    num_loc = B * i * j
    dist_t = jnp.transpose(distance, (0, 1, 2, 4, 3)).reshape(num_loc, D, K)
    sims2d = similarities.reshape(num_loc, K)
    sc_out = _build_sc_kernel(num_loc, K, D, _SC_SHARE)(dist_t, sims2d)
    tc_out = _tc_kernel(dist_t, sims2d, _SC_SHARE)
    out_t = jnp.concatenate(
        [sc_out.reshape(D + 1, _SC_SHARE), tc_out], axis=1
    )
    return out_t.reshape(D + 1, B, i * j).transpose(1, 2, 0)


# speedup vs baseline: 1.0217x; 1.0127x over previous
import functools

import jax
import jax.numpy as jnp
from jax import lax
from jax.experimental import pallas as pl
from jax.experimental.pallas import tpu as pltpu
from jax.experimental.pallas import tpu_sc as plsc

_NUM_CORES = 2
_NUM_SUBCORES = 16
_NUM_WORKERS = _NUM_CORES * _NUM_SUBCORES
_LANES = 16
_CHUNK = 16
_SC_SHARE = 7168
_TC_BLK = 512


def _build_sc_kernel(num_loc, K, D, sc_loc):
    per_w = sc_loc // _NUM_WORKERS
    n_chunks = per_w // _CHUNK
    out_row = D + 1

    mesh = plsc.VectorSubcoreMesh(core_axis_name="c", subcore_axis_name="s")

    @functools.partial(
        pl.kernel,
        mesh=mesh,
        compiler_params=pltpu.CompilerParams(needs_layout_passes=False),
        out_type=jax.ShapeDtypeStruct((out_row * sc_loc,), jnp.float32),
        scratch_types=[
            pltpu.VMEM((per_w, K), jnp.float32),
            pltpu.VMEM((per_w,), jnp.int32),
            pltpu.VMEM((_CHUNK, D, K), jnp.float32),
            pltpu.VMEM((_CHUNK, D, K), jnp.float32),
            pltpu.VMEM((out_row * per_w,), jnp.float32),
            pltpu.SemaphoreType.DMA,
            pltpu.SemaphoreType.DMA,
        ],
    )
    def body(dist_hbm, sims_hbm, out_hbm, sims_v, kbuf_v, db0, db1, outbuf_v,
             sem0, sem1):
        wid = lax.axis_index("s") * _NUM_CORES + lax.axis_index("c")
        base_loc = wid * per_w
        iota = lax.iota(jnp.int32, _LANES)
        dbufs = (db0, db1)
        sems = (sem0, sem1)

        def start(c):
            return pltpu.async_copy(
                dist_hbm.at[pl.ds(base_loc + c * _CHUNK, _CHUNK)],
                dbufs[c % 2],
                sems[c % 2],
            )

        pending = start(0)

        pltpu.sync_copy(sims_hbm.at[pl.ds(base_loc, per_w)], sims_v)

        def group_body(g, carry):
            l0 = g * _LANES + iota
            best_val = plsc.load_gather(
                sims_v, [l0, jnp.zeros((_LANES,), jnp.int32)]
            )
            best_k = jnp.zeros((_LANES,), jnp.int32)
            for k in range(1, K):
                v = plsc.load_gather(
                    sims_v, [l0, jnp.full((_LANES,), k, jnp.int32)]
                )
                take = v > best_val
                best_val = jnp.where(take, v, best_val)
                best_k = jnp.where(take, k, best_k)
            plsc.store_scatter(kbuf_v, [l0], best_k)
            outbuf_v[pl.ds(D * per_w + g * _LANES, _LANES)] = (
                best_k.astype(jnp.float32)
            )
            return carry

        lax.fori_loop(0, per_w // _LANES, group_body, 0)

        for c in range(n_chunks):
            nxt = start(c + 1) if c + 1 < n_chunks else None
            pending.wait()
            dbuf = dbufs[c % 2]
            for g in range(_CHUNK // _LANES):
                lb = c * _CHUNK + g * _LANES
                ks = kbuf_v[pl.ds(lb, _LANES)]
                jvec = g * _LANES + iota
                for dd in range(D):
                    val = plsc.load_gather(
                        dbuf, [jvec, jnp.full((_LANES,), dd, jnp.int32), ks]
                    )
                    outbuf_v[pl.ds(dd * per_w + lb, _LANES)] = val
            pending = nxt

        for comp in range(out_row):
            pltpu.sync_copy(
                outbuf_v.at[pl.ds(comp * per_w, per_w)],
                out_hbm.at[pl.ds(comp * sc_loc + base_loc, per_w)],
            )

    return body


def _tc_body(K, D, d_ref, s_ref, o_ref):
    s = s_ref[...]
    ik = lax.broadcasted_iota(jnp.int32, s.shape, 1)
    m = jnp.max(s, axis=-1, keepdims=True)
    am = jnp.min(jnp.where(s == m, ik, K), axis=-1)
    onehot = (ik == am[:, None]).astype(jnp.float32)
    d = d_ref[...]
    resid = jnp.sum(d * onehot[:, None, :], axis=-1)
    o_ref[...] = jnp.concatenate(
        [resid.T, am[None, :].astype(jnp.float32)], axis=0
    )


def _tc_kernel(dist_t, sims2d, start_loc):
    num_loc, D, K = dist_t.shape
    n = num_loc - start_loc
    off = start_loc // _TC_BLK
    return pl.pallas_call(
        functools.partial(_tc_body, K, D),
        grid=(n // _TC_BLK,),
        in_specs=[
            pl.BlockSpec((_TC_BLK, D, K), lambda g: (g + off, 0, 0)),
            pl.BlockSpec((_TC_BLK, K), lambda g: (g + off, 0)),
        ],
        out_specs=pl.BlockSpec((D + 1, _TC_BLK), lambda g: (0, g)),
        out_shape=jax.ShapeDtypeStruct((D + 1, n), jnp.float32),
    )(dist_t, sims2d)


def kernel(distance, similarities):
    B, i, j, K, D = distance.shape
    num_loc = B * i * j
    dist_t = jnp.transpose(distance, (0, 1, 2, 4, 3)).reshape(num_loc, D, K)
    sims2d = similarities.reshape(num_loc, K)
    sc_out = _build_sc_kernel(num_loc, K, D, _SC_SHARE)(dist_t, sims2d)
    tc_out = _tc_kernel(dist_t, sims2d, _SC_SHARE)
    out_t = jnp.concatenate(
        [sc_out.reshape(D + 1, _SC_SHARE), tc_out], axis=1
    )
    return out_t.reshape(D + 1, B, i * j).transpose(1, 2, 0)
